# baseline (speedup 1.0000x reference)
import functools

import jax
import jax.numpy as jnp
from jax import lax
from jax.experimental import pallas as pl
from jax.experimental.pallas import tpu as pltpu
from jax.experimental.pallas import tpu_sc as plsc

_N = 1000000
_L = 16
_NC = 2
_NS = 16
_NW = _NC * _NS
_VECS = _N // _L
_VPW = _VECS // _NW
_REM = _VECS - _VPW * _NW
_MAXV = _VPW + 1


_PV = 279
_NP = _VPW // _PV


def _focal_body(x_hbm, t_hbm, a_hbm, part_hbm, out_hbm, x0_v, x1_v,
                t_v, a_v, stage_v, red_v, sem0, sem1, sem2, sem3):
    cid = lax.axis_index("c")
    sid = lax.axis_index("s")
    wid = sid * _NC + cid
    sems = [sem0, sem1, sem2]

    rb = wid * _VPW * _L

    def _start_piece(p):
        off = p * _PV * _L
        sem = sems[p % 3]
        return [
            pltpu.async_copy(x_hbm.at[pl.ds(rb + off, _PV * _L)],
                             x0_v.at[pl.ds(off, _PV * _L)], sem),
            pltpu.async_copy(x_hbm.at[pl.ds(_N + rb + off, _PV * _L)],
                             x1_v.at[pl.ds(off, _PV * _L)], sem),
            pltpu.async_copy(t_hbm.at[pl.ds(rb + off, _PV * _L)],
                             t_v.at[pl.ds(off, _PV * _L)], sem),
        ]

    handles = _start_piece(0)

    eb = (_VECS - _REM + lax.rem(wid, _REM)) * _L
    ha = pltpu.async_copy(a_hbm, a_v, sem3)
    tail_handles = [
        pltpu.async_copy(x_hbm.at[pl.ds(eb, _L)],
                         x0_v.at[pl.ds(_VPW * _L, _L)], sem3),
        pltpu.async_copy(x_hbm.at[pl.ds(_N + eb, _L)],
                         x1_v.at[pl.ds(_VPW * _L, _L)], sem3),
        pltpu.async_copy(t_hbm.at[pl.ds(eb, _L)],
                         t_v.at[pl.ds(_VPW * _L, _L)], sem3),
    ]

    tail_on = (wid < _REM).astype(jnp.float32)
    zeros = jnp.zeros((_L,), jnp.int32)
    ha.wait()
    a0 = plsc.load_gather(a_v, [zeros])
    a1 = plsc.load_gather(a_v, [zeros + 1])

    def _term(vec_off, gate):
        t = t_v[pl.ds(vec_off, _L)]
        x0 = x0_v[pl.ds(vec_off, _L)]
        x1 = x1_v[pl.ds(vec_off, _L)]
        pick0 = t == 0
        x = jnp.where(pick0, x0, x1)
        a = jnp.where(pick0, a0, a1) * gate
        nonneg = x >= 0
        u = jnp.exp(jnp.minimum(x, -x))
        z = 1.0 + u
        zp1 = z + 1.0
        r = 1.0 / (z * zp1)
        d = r * zp1
        s = u * (r * z)
        s2 = s * s
        p = jnp.float32(1 / 9)
        p = jnp.float32(1 / 7) + s2 * p
        p = jnp.float32(1 / 5) + s2 * p
        p = jnp.float32(1 / 3) + s2 * p
        p = jnp.float32(1.0) + s2 * p
        neg_logp = 2.0 * s * p - jnp.where(nonneg, jnp.float32(0.0), x)
        omp = jnp.where(nonneg, u * d, d)
        return a * (omp * omp) * neg_logp

    one = jnp.float32(1.0)
    zf = jnp.zeros((_L,), jnp.float32)
    accs = (zf, zf, zf)
    for piece in range(_NP):
        for h in handles:
            h.wait()
        if piece + 1 < _NP:
            handles = _start_piece(piece + 1)
        base = piece * _PV * _L

        def _iter(j, accs, base=base):
            o = base + j * (3 * _L)
            return (accs[0] + _term(o, one),
                    accs[1] + _term(o + _L, one),
                    accs[2] + _term(o + 2 * _L, one))

        accs = lax.fori_loop(0, _PV // 3, _iter, accs)

    for h in tail_handles:
        h.wait()
    acc = accs[0] + accs[1] + accs[2] + _term(_VPW * _L, tail_on)

    stage_v[...] = acc
    pltpu.sync_copy(stage_v, part_hbm.at[cid, sid])
    plsc.subcore_barrier()

    @pl.when(sid == 0)
    def _reduce():
        pltpu.sync_copy(part_hbm.at[cid], red_v)
        tot = red_v[0]
        for j in range(1, _NS):
            tot = tot + red_v[j]
        total = jnp.sum(tot) * jnp.float32(1.0 / _N)
        stage_v[...] = jnp.full((_L,), total, jnp.float32)
        pltpu.sync_copy(stage_v, out_hbm.at[cid])


def kernel(inputs, targets, alpha):
    mesh = plsc.VectorSubcoreMesh(core_axis_name="c", subcore_axis_name="s")
    f = pl.kernel(
        _focal_body,
        out_type=(jax.ShapeDtypeStruct((_NC, _NS, _L), jnp.float32),
                  jax.ShapeDtypeStruct((_NC, _L), jnp.float32)),
        mesh=mesh,
        scratch_types=[
            pltpu.VMEM((_MAXV * _L,), jnp.float32),
            pltpu.VMEM((_MAXV * _L,), jnp.float32),
            pltpu.VMEM((_MAXV * _L,), jnp.int32),
            pltpu.VMEM((2,), jnp.float32),
            pltpu.VMEM((_L,), jnp.float32),
            pltpu.VMEM((_NS, _L), jnp.float32),
            pltpu.SemaphoreType.DMA,
            pltpu.SemaphoreType.DMA,
            pltpu.SemaphoreType.DMA,
            pltpu.SemaphoreType.DMA,
        ],
        compiler_params=pltpu.CompilerParams(needs_layout_passes=False),
    )
    x01 = jnp.transpose(inputs).ravel()
    _, out = f(x01, targets, alpha.reshape(-1))
    return out[0, 0] + out[1, 0]


# speedup vs baseline: 1.0041x; 1.0041x over previous
import jax
import jax.numpy as jnp
from jax import lax
from jax.experimental import pallas as pl
from jax.experimental.pallas import tpu as pltpu
from jax.experimental.pallas import tpu_sc as plsc

_N = 1000000
_L = 16
_NC = 2
_NS = 16
_NW = _NC * _NS
_VECS = _N // _L
_VPW = _VECS // _NW
_REM = _VECS - _VPW * _NW
_MAXV = _VPW + 1


_PV = 651
_NP = _VPW // _PV


def _focal_body(x_hbm, t_hbm, a_hbm, part_hbm, out_hbm, x0_v, x1_v,
                t_v, a_v, stage_v, red_v, sem0, sem1, sem2, sem3):
    cid = lax.axis_index("c")
    sid = lax.axis_index("s")
    wid = sid * _NC + cid
    sems = [sem0, sem1, sem2]

    rb = wid * _VPW * _L

    def _start_piece(p):
        off = p * _PV * _L
        sem = sems[p % 3]
        return [
            pltpu.async_copy(x_hbm.at[pl.ds(rb + off, _PV * _L)],
                             x0_v.at[pl.ds(off, _PV * _L)], sem),
            pltpu.async_copy(x_hbm.at[pl.ds(_N + rb + off, _PV * _L)],
                             x1_v.at[pl.ds(off, _PV * _L)], sem),
            pltpu.async_copy(t_hbm.at[pl.ds(rb + off, _PV * _L)],
                             t_v.at[pl.ds(off, _PV * _L)], sem),
        ]

    handles = _start_piece(0)

    eb = (_VECS - _REM + lax.rem(wid, _REM)) * _L
    ha = pltpu.async_copy(a_hbm, a_v, sem3)
    tail_handles = [
        pltpu.async_copy(x_hbm.at[pl.ds(eb, _L)],
                         x0_v.at[pl.ds(_VPW * _L, _L)], sem3),
        pltpu.async_copy(x_hbm.at[pl.ds(_N + eb, _L)],
                         x1_v.at[pl.ds(_VPW * _L, _L)], sem3),
        pltpu.async_copy(t_hbm.at[pl.ds(eb, _L)],
                         t_v.at[pl.ds(_VPW * _L, _L)], sem3),
    ]

    tail_on = (wid < _REM).astype(jnp.float32)
    zeros = jnp.zeros((_L,), jnp.int32)
    ha.wait()
    a0 = plsc.load_gather(a_v, [zeros])
    a1 = plsc.load_gather(a_v, [zeros + 1])

    def _term(vec_off, gate):
        t = t_v[pl.ds(vec_off, _L)]
        x0 = x0_v[pl.ds(vec_off, _L)]
        x1 = x1_v[pl.ds(vec_off, _L)]
        pick0 = t == 0
        x = jnp.where(pick0, x0, x1)
        a = jnp.where(pick0, a0, a1) * gate
        nonneg = x >= 0
        u = jnp.exp(jnp.minimum(x, -x))
        z = 1.0 + u
        zp1 = z + 1.0
        r = 1.0 / (z * zp1)
        d = r * zp1
        s = u * (r * z)
        s2 = s * s
        p = jnp.float32(1 / 9)
        p = jnp.float32(1 / 7) + s2 * p
        p = jnp.float32(1 / 5) + s2 * p
        p = jnp.float32(1 / 3) + s2 * p
        p = jnp.float32(1.0) + s2 * p
        neg_logp = 2.0 * s * p - jnp.where(nonneg, jnp.float32(0.0), x)
        omp = jnp.where(nonneg, u * d, d)
        return a * (omp * omp) * neg_logp

    one = jnp.float32(1.0)
    zf = jnp.zeros((_L,), jnp.float32)
    accs = (zf, zf, zf)
    for piece in range(_NP):
        for h in handles:
            h.wait()
        if piece + 1 < _NP:
            handles = _start_piece(piece + 1)
        base = piece * _PV * _L

        def _iter(j, accs, base=base):
            o = base + j * (3 * _L)
            return (accs[0] + _term(o, one),
                    accs[1] + _term(o + _L, one),
                    accs[2] + _term(o + 2 * _L, one))

        accs = lax.fori_loop(0, _PV // 3, _iter, accs)

    for h in tail_handles:
        h.wait()
    acc = accs[0] + accs[1] + accs[2] + _term(_VPW * _L, tail_on)

    stage_v[...] = acc
    pltpu.sync_copy(stage_v, part_hbm.at[cid, sid])
    plsc.subcore_barrier()

    @pl.when(sid == 0)
    def _reduce():
        pltpu.sync_copy(part_hbm.at[cid], red_v)
        tot = red_v[0]
        for j in range(1, _NS):
            tot = tot + red_v[j]
        total = jnp.sum(tot) * jnp.float32(1.0 / _N)
        stage_v[...] = jnp.full((_L,), total, jnp.float32)
        pltpu.sync_copy(stage_v, out_hbm.at[cid])


def kernel(inputs, targets, alpha):
    mesh = plsc.VectorSubcoreMesh(core_axis_name="c", subcore_axis_name="s")
    f = pl.kernel(
        _focal_body,
        out_type=(jax.ShapeDtypeStruct((_NC, _NS, _L), jnp.float32),
                  jax.ShapeDtypeStruct((_NC, _L), jnp.float32)),
        mesh=mesh,
        scratch_types=[
            pltpu.VMEM((_MAXV * _L,), jnp.float32),
            pltpu.VMEM((_MAXV * _L,), jnp.float32),
            pltpu.VMEM((_MAXV * _L,), jnp.int32),
            pltpu.VMEM((2,), jnp.float32),
            pltpu.VMEM((_L,), jnp.float32),
            pltpu.VMEM((_NS, _L), jnp.float32),
            pltpu.SemaphoreType.DMA,
            pltpu.SemaphoreType.DMA,
            pltpu.SemaphoreType.DMA,
            pltpu.SemaphoreType.DMA,
        ],
        compiler_params=pltpu.CompilerParams(needs_layout_passes=False),
    )
    x01 = jnp.transpose(inputs).ravel()
    _, out = f(x01, targets, alpha.reshape(-1))
    return out[0, 0] + out[1, 0]
